# baseline (speedup 1.0000x reference)
import functools

import jax
import jax.numpy as jnp
from jax.experimental import pallas as pl
from jax.experimental.pallas import tpu as pltpu

_STRIDE = 16.0
_CLS_THRESH = 0.95
_MAX_IOU = 0.1
_NUM_PROPOSALS = 10
_NUM_ANCHORS = 9
_GRP = 16


def _body(xa_ref, xb_ref, anchors_ref, out_ref, scr_ref, *, B, H, W, C):
    b = pl.program_id(0)
    half = C // 2

    def t(c):
        return xa_ref[0, c] if c < half else xb_ref[0, c - half]

    a = [t(C - 10 + j) for j in range(_NUM_ANCHORS)]
    m = a[0]
    for j in range(1, _NUM_ANCHORS):
        m = jnp.maximum(m, a[j])
    a_idx = jnp.full((H, W), _NUM_ANCHORS, jnp.int32)
    for j in range(_NUM_ANCHORS - 1, -1, -1):
        a_idx = jnp.where(a[j] == m, j, a_idx)

    selj = [a_idx == j for j in range(_NUM_ANCHORS)]
    d = []
    for k in range(4):
        acc = t(k)
        for j in range(1, _NUM_ANCHORS):
            acc = jnp.where(selj[j], t(4 * j + k), acc)
        d.append(acc)
    aw = jnp.zeros((H, W), jnp.float32)
    ah = jnp.zeros((H, W), jnp.float32)
    for j in range(_NUM_ANCHORS):
        wj = anchors_ref[j, 1]
        rj = anchors_ref[j, 0]
        aw = jnp.where(selj[j], wj, aw)
        ah = jnp.where(selj[j], wj / rj, ah)

    yi = jax.lax.broadcasted_iota(jnp.int32, (H, W), 0)
    xi = jax.lax.broadcasted_iota(jnp.int32, (H, W), 1)
    border = (yi == 0) | (yi == H - 1) | (xi == 0) | (xi == W - 1)
    sc = jnp.where(border, 0.0, t(C - 1))
    ax = (xi.astype(jnp.float32) + 0.5) * _STRIDE
    ay = (yi.astype(jnp.float32) + 0.5) * _STRIDE

    cx = d[0] * aw + ax
    cy = d[1] * ah + ay
    bw = jnp.exp(d[2]) * aw
    bh = jnp.exp(d[3]) * ah
    y1 = cy - bh / 2.0
    x1 = cx - bw / 2.0
    y2 = cy + bh / 2.0
    x2 = cx + bw / 2.0

    neg_inf = jnp.float32(-jnp.inf)
    areab = jnp.maximum(0.0, y2 - y1) * jnp.maximum(0.0, x2 - x1)
    planes = [jnp.where(sc > _CLS_THRESH, sc, neg_inf),
              y1, x1, y2, x2, cx, cy, bw, bh, areab]
    for k in range(10):
        scr_ref[k, b] = planes[k]

    G = _GRP
    @pl.when(b % G == G - 1)
    def _nms():
        lo = b - (G - 1)
        sel4 = pl.ds(lo, G)
        msk0 = scr_ref[0, sel4]
        py1 = scr_ref[1, sel4]
        px1 = scr_ref[2, sel4]
        py2 = scr_ref[3, sel4]
        px2 = scr_ref[4, sel4]
        pcx = scr_ref[5, sel4]
        pcy = scr_ref[6, sel4]
        pbw = scr_ref[7, sel4]
        pbh = scr_ref[8, sel4]
        area_b = scr_ref[9, sel4]
        N = H * W
        lin = (yi * W + xi)[None]
        out0 = jnp.zeros((G, _NUM_PROPOSALS, 4), jnp.float32)
        rowi = jax.lax.broadcasted_iota(jnp.int32, (G, _NUM_PROPOSALS, 4), 1)
        coli = jax.lax.broadcasted_iota(jnp.int32, (G, _NUM_PROPOSALS, 4), 2)

        def red(x, op):
            r = op(x, axis=2, keepdims=True)
            return op(r, axis=1, keepdims=True)

        def body2(i, carry):
            out, masked = carry
            mx = red(masked, jnp.max)
            any_valid = mx > 0.0
            j = red(jnp.where(masked == mx, lin, N), jnp.min)
            selp = lin == j

            def pick(plane):
                return red(jnp.where(selp, plane, 0.0), jnp.sum)

            bcx = pick(pcx); bcy = pick(pcy); bbw = pick(pbw); bbh = pick(pbh)
            by1 = bcy - bbh / 2.0
            bx1 = bcx - bbw / 2.0
            by2 = bcy + bbh / 2.0
            bx2 = bcx + bbw / 2.0

            iy1 = jnp.maximum(by1, py1)
            ix1 = jnp.maximum(bx1, px1)
            iy2 = jnp.minimum(by2, py2)
            ix2 = jnp.minimum(bx2, px2)
            inter = (jnp.maximum(0.0, iy2 - iy1)
                     * jnp.maximum(0.0, ix2 - ix1))
            area_a = (jnp.maximum(0.0, by2 - by1)
                      * jnp.maximum(0.0, bx2 - bx1))
            union = area_a + area_b - inter
            iou = jnp.where(union > 0.0, inter / union, 0.0)

            supp = ((iou > _MAX_IOU) | (lin == j)) & any_valid
            masked = jnp.where(supp, neg_inf, masked)

            vals = jnp.where(coli == 0, bcx, jnp.where(coli == 1, bcy,
                             jnp.where(coli == 2, bbw, bbh)))
            upd = (rowi == i) & any_valid
            out = jnp.where(upd, vals, out)
            return out, masked

        out, _ = jax.lax.fori_loop(0, _NUM_PROPOSALS, body2, (out0, msk0))
        out_ref[sel4] = out


@jax.jit
def kernel(inputs, anchors):
    B, H, W, C = inputs.shape
    xp = inputs.transpose(0, 3, 1, 2)

    return pl.pallas_call(
        functools.partial(_body, B=B, H=H, W=W, C=C),
        grid=(B,),
        in_specs=[
            pl.BlockSpec((1, C // 2, H, W), lambda b: (b, 0, 0, 0)),
            pl.BlockSpec((1, C // 2, H, W), lambda b: (b, 1, 0, 0)),
            pl.BlockSpec(memory_space=pltpu.SMEM),
        ],
        out_specs=pl.BlockSpec((B, _NUM_PROPOSALS, 4), lambda b: (0, 0, 0)),
        out_shape=jax.ShapeDtypeStruct((B, _NUM_PROPOSALS, 4), jnp.float32),
        scratch_shapes=[pltpu.VMEM((10, B, H, W), jnp.float32)],
    )(xp, xp, anchors)


# speedup vs baseline: 1.0019x; 1.0019x over previous
import functools

import jax
import jax.numpy as jnp
from jax.experimental import pallas as pl
from jax.experimental.pallas import tpu as pltpu

_STRIDE = 16.0
_CLS_THRESH = 0.95
_MAX_IOU = 0.1
_NUM_PROPOSALS = 10
_NUM_ANCHORS = 9
_GRP = 16


def _body(x_ref, anchors_ref, out_ref, scr_ref, *, B, H, W, C):
    b = pl.program_id(0)

    def t(c):
        return x_ref[0, c]

    a = [t(C - 10 + j) for j in range(_NUM_ANCHORS)]
    m = a[0]
    for j in range(1, _NUM_ANCHORS):
        m = jnp.maximum(m, a[j])
    a_idx = jnp.full((H, W), _NUM_ANCHORS, jnp.int32)
    for j in range(_NUM_ANCHORS - 1, -1, -1):
        a_idx = jnp.where(a[j] == m, j, a_idx)

    selj = [a_idx == j for j in range(_NUM_ANCHORS)]
    d = []
    for k in range(4):
        acc = t(k)
        for j in range(1, _NUM_ANCHORS):
            acc = jnp.where(selj[j], t(4 * j + k), acc)
        d.append(acc)
    aw = jnp.zeros((H, W), jnp.float32)
    ah = jnp.zeros((H, W), jnp.float32)
    for j in range(_NUM_ANCHORS):
        wj = anchors_ref[j, 1]
        rj = anchors_ref[j, 0]
        aw = jnp.where(selj[j], wj, aw)
        ah = jnp.where(selj[j], wj / rj, ah)

    yi = jax.lax.broadcasted_iota(jnp.int32, (H, W), 0)
    xi = jax.lax.broadcasted_iota(jnp.int32, (H, W), 1)
    border = (yi == 0) | (yi == H - 1) | (xi == 0) | (xi == W - 1)
    sc = jnp.where(border, 0.0, t(C - 1))
    ax = (xi.astype(jnp.float32) + 0.5) * _STRIDE
    ay = (yi.astype(jnp.float32) + 0.5) * _STRIDE

    cx = d[0] * aw + ax
    cy = d[1] * ah + ay
    bw = jnp.exp(d[2]) * aw
    bh = jnp.exp(d[3]) * ah
    y1 = cy - bh / 2.0
    x1 = cx - bw / 2.0
    y2 = cy + bh / 2.0
    x2 = cx + bw / 2.0

    neg_inf = jnp.float32(-jnp.inf)
    areab = jnp.maximum(0.0, y2 - y1) * jnp.maximum(0.0, x2 - x1)
    planes = [jnp.where(sc > _CLS_THRESH, sc, neg_inf),
              y1, x1, y2, x2, cx, cy, bw, bh, areab]
    for k in range(10):
        scr_ref[k, b] = planes[k]

    G = _GRP
    @pl.when(b % G == G - 1)
    def _nms():
        lo = b - (G - 1)
        sel4 = pl.ds(lo, G)
        msk0 = scr_ref[0, sel4]
        py1 = scr_ref[1, sel4]
        px1 = scr_ref[2, sel4]
        py2 = scr_ref[3, sel4]
        px2 = scr_ref[4, sel4]
        pcx = scr_ref[5, sel4]
        pcy = scr_ref[6, sel4]
        pbw = scr_ref[7, sel4]
        pbh = scr_ref[8, sel4]
        area_b = scr_ref[9, sel4]
        N = H * W
        lin = (yi * W + xi)[None]
        out0 = jnp.zeros((G, _NUM_PROPOSALS, 4), jnp.float32)
        rowi = jax.lax.broadcasted_iota(jnp.int32, (G, _NUM_PROPOSALS, 4), 1)
        coli = jax.lax.broadcasted_iota(jnp.int32, (G, _NUM_PROPOSALS, 4), 2)

        def red(x, op):
            r = op(x, axis=2, keepdims=True)
            return op(r, axis=1, keepdims=True)

        def body2(i, carry):
            out, masked = carry
            mx = red(masked, jnp.max)
            any_valid = mx > 0.0
            j = red(jnp.where(masked == mx, lin, N), jnp.min)
            selp = lin == j

            def pick(plane):
                return red(jnp.where(selp, plane, 0.0), jnp.sum)

            bcx = pick(pcx); bcy = pick(pcy); bbw = pick(pbw); bbh = pick(pbh)
            by1 = bcy - bbh / 2.0
            bx1 = bcx - bbw / 2.0
            by2 = bcy + bbh / 2.0
            bx2 = bcx + bbw / 2.0

            iy1 = jnp.maximum(by1, py1)
            ix1 = jnp.maximum(bx1, px1)
            iy2 = jnp.minimum(by2, py2)
            ix2 = jnp.minimum(bx2, px2)
            inter = (jnp.maximum(0.0, iy2 - iy1)
                     * jnp.maximum(0.0, ix2 - ix1))
            area_a = (jnp.maximum(0.0, by2 - by1)
                      * jnp.maximum(0.0, bx2 - bx1))
            union = area_a + area_b - inter
            iou = jnp.where(union > 0.0, inter / union, 0.0)

            supp = ((iou > _MAX_IOU) | (lin == j)) & any_valid
            masked = jnp.where(supp, neg_inf, masked)

            vals = jnp.where(coli == 0, bcx, jnp.where(coli == 1, bcy,
                             jnp.where(coli == 2, bbw, bbh)))
            upd = (rowi == i) & any_valid
            out = jnp.where(upd, vals, out)
            return out, masked

        out, _ = jax.lax.fori_loop(0, _NUM_PROPOSALS, body2, (out0, msk0))
        out_ref[sel4] = out


@jax.jit
def kernel(inputs, anchors):
    B, H, W, C = inputs.shape
    xp = inputs.transpose(0, 3, 1, 2)

    return pl.pallas_call(
        functools.partial(_body, B=B, H=H, W=W, C=C),
        grid=(B,),
        in_specs=[
            pl.BlockSpec((1, C, H, W), lambda b: (b, 0, 0, 0)),
            pl.BlockSpec(memory_space=pltpu.SMEM),
        ],
        out_specs=pl.BlockSpec((B, _NUM_PROPOSALS, 4), lambda b: (0, 0, 0)),
        out_shape=jax.ShapeDtypeStruct((B, _NUM_PROPOSALS, 4), jnp.float32),
        scratch_shapes=[pltpu.VMEM((10, B, H, W), jnp.float32)],
    )(xp, anchors)
